# Initial kernel scaffold; baseline (speedup 1.0000x reference)
import jax
import jax.numpy as jnp
from jax.experimental import pallas as pl


def kernel(x, weight, bias):
    raise NotImplementedError("write your pallas kernel here")


# speedup vs baseline: 1.0354x; 1.0354x over previous
import functools

import jax
import jax.numpy as jnp
from jax import lax
from jax.experimental import pallas as pl

TOPK = 8
NG = 64
D = 2048


def _gate_block(x_ref, w_ref, b_ref, idx_ref, wgt_ref):
    logits = jnp.dot(x_ref[...], w_ref[...], preferred_element_type=jnp.float32)
    logits = logits + b_ref[...]
    br = logits.shape[0]
    e = jnp.exp(logits - jnp.max(logits, axis=1, keepdims=True))
    scores = e / jnp.sum(e, axis=1, keepdims=True)
    col = lax.broadcasted_iota(jnp.int32, (br, NG), 1)
    vals = []
    idxs = []
    cur = scores
    for _ in range(TOPK):
        m = jnp.max(cur, axis=1, keepdims=True)
        a = jnp.min(jnp.where(cur == m, col, NG), axis=1, keepdims=True)
        vals.append(m)
        idxs.append(a)
        cur = jnp.where(col == a, -1.0, cur)
    v = jnp.concatenate(vals, axis=1)
    i = jnp.concatenate(idxs, axis=1)
    w = v / (jnp.sum(v, axis=1, keepdims=True) + 1e-20)
    idx_ref[...] = i
    wgt_ref[...] = w


@functools.partial(jax.jit, static_argnames=("br",))
def _gate_tc(hs, wt, b2, br=512):
    rows = hs.shape[0]
    grid = (rows // br,)
    return pl.pallas_call(
        _gate_block,
        grid=grid,
        in_specs=[
            pl.BlockSpec((br, D), lambda ii: (ii, 0)),
            pl.BlockSpec((D, NG), lambda ii: (0, 0)),
            pl.BlockSpec((1, NG), lambda ii: (0, 0)),
        ],
        out_specs=[
            pl.BlockSpec((br, TOPK), lambda ii: (ii, 0)),
            pl.BlockSpec((br, TOPK), lambda ii: (ii, 0)),
        ],
        out_shape=[
            jax.ShapeDtypeStruct((rows, TOPK), jnp.int32),
            jax.ShapeDtypeStruct((rows, TOPK), jnp.float32),
        ],
    )(hs, wt, b2)


def kernel(x, weight, bias):
    bsz, seq_len, h = x.shape
    hs = x.reshape(-1, h)
    wt = weight.T
    b2 = bias.reshape(1, NG)
    idx, wgt = _gate_tc(hs, wt, b2)
    aux_loss = jnp.zeros((), dtype=jnp.float32)
    return (idx, wgt, aux_loss)
